# Initial kernel scaffold; baseline (speedup 1.0000x reference)
import functools

import jax
import jax.numpy as jnp
from jax import lax
from jax.experimental import pallas as pl
from jax.experimental.pallas import tpu as pltpu
from jax.experimental.pallas import tpu_sc as plsc

N_NODES_ = 10000
N_EDGES_ = 320000
D_ = 128
L_ = 16
NW_ = 32
E_PER_W = N_EDGES_ // NW_
CHUNK = 400
N_CHUNKS = E_PER_W // CHUNK


def _body(z_src, z_dst, src_idx, dst_idx, out,
          sidx_v, didx_v, srows_v, drows_v, out_v,
          sem_s, sem_d):
  wid = lax.axis_index("s") * 2 + lax.axis_index("c")
  base_w = wid * E_PER_W

  def chunk_body(j, _):
    base = base_w + j * CHUNK
    pltpu.sync_copy(src_idx.at[pl.ds(base, CHUNK)], sidx_v)
    pltpu.sync_copy(dst_idx.at[pl.ds(base, CHUNK)], didx_v)
    cp_s = pltpu.async_copy(z_src.at[sidx_v], srows_v, sem_s)
    cp_d = pltpu.async_copy(z_dst.at[didx_v], drows_v, sem_d)
    cp_s.wait()
    cp_d.wait()

    def group_body(g, _):
      rows = g * L_ + lax.iota(jnp.int32, (L_,))
      acc = jnp.zeros((L_,), jnp.float32)

      def d_body(db, acc):
        for k in range(16):
          d = jnp.full((L_,), db * 16 + k, jnp.int32)
          s = plsc.load_gather(srows_v, [rows, d])
          t = plsc.load_gather(drows_v, [rows, d])
          acc = acc + s * t
        return acc

      acc = lax.fori_loop(0, D_ // 16, d_body, acc)
      out_v[pl.ds(g * L_, L_)] = acc
      return 0

    lax.fori_loop(0, CHUNK // L_, group_body, 0)
    pltpu.sync_copy(out_v, out.at[pl.ds(base, CHUNK)])
    return 0

  lax.fori_loop(0, N_CHUNKS, chunk_body, 0)


@jax.jit
def _decoder(z_src, z_dst, src_idx, dst_idx):
  mesh = plsc.VectorSubcoreMesh(core_axis_name="c", subcore_axis_name="s")
  return pl.kernel(
      _body,
      out_type=jax.ShapeDtypeStruct((N_EDGES_,), jnp.float32),
      mesh=mesh,
      scratch_types=[
          pltpu.VMEM((CHUNK,), jnp.int32),
          pltpu.VMEM((CHUNK,), jnp.int32),
          pltpu.VMEM((CHUNK, D_), jnp.float32),
          pltpu.VMEM((CHUNK, D_), jnp.float32),
          pltpu.VMEM((CHUNK,), jnp.float32),
          pltpu.SemaphoreType.DMA,
          pltpu.SemaphoreType.DMA,
      ],
  )(z_src, z_dst, src_idx, dst_idx)


def kernel(z_src, z_dst, edge_index):
  src_idx = edge_index[0].astype(jnp.int32)
  dst_idx = edge_index[1].astype(jnp.int32)
  return _decoder(z_src, z_dst, src_idx, dst_idx)


# speedup vs baseline: 3.4317x; 3.4317x over previous
import functools

import jax
import jax.numpy as jnp
from jax import lax
from jax.experimental import pallas as pl
from jax.experimental.pallas import tpu as pltpu
from jax.experimental.pallas import tpu_sc as plsc

N_NODES_ = 10000
N_EDGES_ = 320000
D_ = 128
L_ = 16
NW_ = 32
E_PER_W = N_EDGES_ // NW_
CHUNK = 400
N_CHUNKS = E_PER_W // CHUNK


def _body(z_src, z_dst, src_idx, dst_idx, out,
          sidx_v, didx_v, srows_v, drows_v, out_v,
          sem_s, sem_d):
  wid = lax.axis_index("s") * 2 + lax.axis_index("c")
  base_w = wid * E_PER_W

  def chunk_body(j, _):
    base = base_w + j * CHUNK
    pltpu.sync_copy(src_idx.at[pl.ds(base, CHUNK)], sidx_v)
    pltpu.sync_copy(dst_idx.at[pl.ds(base, CHUNK)], didx_v)
    cp_s = pltpu.async_copy(z_src.at[sidx_v], srows_v, sem_s)
    cp_d = pltpu.async_copy(z_dst.at[didx_v], drows_v, sem_d)
    cp_s.wait()
    cp_d.wait()

    lanes = lax.iota(jnp.int32, L_)

    @plsc.parallel_loop(0, CHUNK // L_)
    def g_body(g):
      vals = jnp.zeros((L_,), jnp.float32)
      for l in range(L_):
        e = g * L_ + l
        acc = srows_v[e, pl.ds(0, L_)] * drows_v[e, pl.ds(0, L_)]
        for db in range(1, D_ // L_):
          acc = acc + srows_v[e, pl.ds(db * L_, L_)] * drows_v[e, pl.ds(db * L_, L_)]
        vals = jnp.where(lanes == l, jnp.sum(acc), vals)
      out_v[pl.ds(g * L_, L_)] = vals
    pltpu.sync_copy(out_v, out.at[pl.ds(base, CHUNK)])
    return 0

  lax.fori_loop(0, N_CHUNKS, chunk_body, 0)


@jax.jit
def _decoder(z_src, z_dst, src_idx, dst_idx):
  mesh = plsc.VectorSubcoreMesh(core_axis_name="c", subcore_axis_name="s")
  return pl.kernel(
      _body,
      out_type=jax.ShapeDtypeStruct((N_EDGES_,), jnp.float32),
      mesh=mesh,
      compiler_params=pltpu.CompilerParams(needs_layout_passes=False),
      scratch_types=[
          pltpu.VMEM((CHUNK,), jnp.int32),
          pltpu.VMEM((CHUNK,), jnp.int32),
          pltpu.VMEM((CHUNK, D_), jnp.float32),
          pltpu.VMEM((CHUNK, D_), jnp.float32),
          pltpu.VMEM((CHUNK,), jnp.float32),
          pltpu.SemaphoreType.DMA,
          pltpu.SemaphoreType.DMA,
      ],
  )(z_src, z_dst, src_idx, dst_idx)


def kernel(z_src, z_dst, edge_index):
  src_idx = edge_index[0].astype(jnp.int32)
  dst_idx = edge_index[1].astype(jnp.int32)
  return _decoder(z_src, z_dst, src_idx, dst_idx)


# speedup vs baseline: 4.0575x; 1.1823x over previous
import jax
import jax.numpy as jnp
from jax import lax
from jax.experimental import pallas as pl
from jax.experimental.pallas import tpu as pltpu
from jax.experimental.pallas import tpu_sc as plsc

N_EDGES_ = 320000
D_ = 128
L_ = 16
NW_ = 32
E_PER_W = N_EDGES_ // NW_
CHUNK = 80
N_CHUNKS = E_PER_W // CHUNK


def _body(z_src, z_dst, src_idx, dst_idx, out,
          sidx_v, didx_v, out_v, sbuf0, dbuf0, sbuf1, dbuf1,
          sem_s0, sem_d0, sem_s1, sem_d1):
  wid = lax.axis_index("s") * 2 + lax.axis_index("c")
  base_w = wid * E_PER_W

  pltpu.sync_copy(src_idx.at[pl.ds(base_w, E_PER_W)], sidx_v)
  pltpu.sync_copy(dst_idx.at[pl.ds(base_w, E_PER_W)], didx_v)

  bufs = ((sbuf0, dbuf0, sem_s0, sem_d0), (sbuf1, dbuf1, sem_s1, sem_d1))
  lanes = lax.iota(jnp.int32, L_)

  def fire(c, p):
    sb, db, ss, sd = bufs[p]
    pltpu.async_copy(z_src.at[sidx_v.at[pl.ds(c * CHUNK, CHUNK)]], sb, ss)
    pltpu.async_copy(z_dst.at[didx_v.at[pl.ds(c * CHUNK, CHUNK)]], db, sd)

  def wait(p):
    sb, db, ss, sd = bufs[p]
    pltpu.make_async_copy(z_src.at[pl.ds(0, CHUNK)], sb, ss).wait()
    pltpu.make_async_copy(z_dst.at[pl.ds(0, CHUNK)], db, sd).wait()

  def compute(c, p):
    sb, db, _, _ = bufs[p]

    @plsc.parallel_loop(0, CHUNK // L_)
    def g_body(g):
      vals = jnp.zeros((L_,), jnp.float32)
      for l in range(L_):
        e = g * L_ + l
        acc = sb[e, pl.ds(0, L_)] * db[e, pl.ds(0, L_)]
        for d in range(1, D_ // L_):
          acc = acc + sb[e, pl.ds(d * L_, L_)] * db[e, pl.ds(d * L_, L_)]
        vals = jnp.where(lanes == l, jnp.sum(acc), vals)
      out_v[pl.ds(c * CHUNK + g * L_, L_)] = vals

  fire(0, 0)

  def pair_body(k, _):
    c = 2 * k + 1
    fire(c, 1)
    wait(0)
    compute(c - 1, 0)
    fire(c + 1, 0)
    wait(1)
    compute(c, 1)
    return 0

  lax.fori_loop(0, (N_CHUNKS - 1) // 2, pair_body, 0)
  wait(0)
  compute(N_CHUNKS - 1, 0)

  pltpu.sync_copy(out_v, out.at[pl.ds(base_w, E_PER_W)])


@jax.jit
def _decoder(z_src, z_dst, src_idx, dst_idx):
  mesh = plsc.VectorSubcoreMesh(core_axis_name="c", subcore_axis_name="s")
  return pl.kernel(
      _body,
      out_type=jax.ShapeDtypeStruct((N_EDGES_,), jnp.float32),
      mesh=mesh,
      compiler_params=pltpu.CompilerParams(needs_layout_passes=False),
      scratch_types=[
          pltpu.VMEM((E_PER_W,), jnp.int32),
          pltpu.VMEM((E_PER_W,), jnp.int32),
          pltpu.VMEM((E_PER_W,), jnp.float32),
          pltpu.VMEM((CHUNK, D_), jnp.float32),
          pltpu.VMEM((CHUNK, D_), jnp.float32),
          pltpu.VMEM((CHUNK, D_), jnp.float32),
          pltpu.VMEM((CHUNK, D_), jnp.float32),
          pltpu.SemaphoreType.DMA,
          pltpu.SemaphoreType.DMA,
          pltpu.SemaphoreType.DMA,
          pltpu.SemaphoreType.DMA,
      ],
  )(z_src, z_dst, src_idx, dst_idx)


def kernel(z_src, z_dst, edge_index):
  src_idx = edge_index[0].astype(jnp.int32)
  dst_idx = edge_index[1].astype(jnp.int32)
  return _decoder(z_src, z_dst, src_idx, dst_idx)
